# baseline (speedup 1.0000x reference)
import functools

import jax
import jax.numpy as jnp
from jax import lax
from jax.experimental import pallas as pl
from jax.experimental.pallas import tpu as pltpu
from jax.experimental.pallas import tpu_sc as plsc

_NUM_CORES = 2
_NUM_SUBCORES = 16
_NW = _NUM_CORES * _NUM_SUBCORES
_LANES = 16

_ROW_BLK = 12800
_NBH = 4
_HALFV = _NBH * _ROW_BLK


def _round_bf16_bits(prod):
    bits = lax.bitcast_convert_type(prod, jnp.int32)
    lsb = lax.bitwise_and(lax.shift_right_logical(bits, 16), 1)
    return lax.shift_right_logical(bits + 0x7FFF + lsb, 16)


def _scores(emb, W, b, x, hist):
    V, D = emb.shape
    B, H = x.shape
    assert V <= 2 * _HALFV and B % _NBH == 0
    xrows = B // _NBH
    inv = 1.0 / float(hist)

    def body(lo_ref, hi_ref, w_ref, b_ref, out_ref):
        dims = (((1,), (1,)), ((), ()))
        lo = lax.dot_general(w_ref[:], lo_ref[:], dims,
                             preferred_element_type=jnp.float32)
        hi = lax.dot_general(w_ref[:], hi_ref[:], dims,
                             preferred_element_type=jnp.float32)
        lo = lo * inv + b_ref[0] * inv
        hi = hi * inv + b_ref[0] * inv
        out_ref[0] = lax.bitwise_or(
            _round_bf16_bits(lo), lax.shift_left(_round_bf16_bits(hi), 16))

    out = pl.pallas_call(
        body,
        grid=(_NBH,),
        in_specs=[
            pl.BlockSpec((_ROW_BLK, D), lambda i: (i, 0)),
            pl.BlockSpec((_ROW_BLK, D), lambda i: (i + _NBH, 0)),
            pl.BlockSpec((1, D), lambda i: (0, 0)),
            pl.BlockSpec(memory_space=pltpu.SMEM),
        ],
        out_specs=pl.BlockSpec((1, 1, _ROW_BLK), lambda i: (i, 0, 0)),
        out_shape=jax.ShapeDtypeStruct((_NBH, 1, _ROW_BLK), jnp.int32),
    )(emb, emb, W, b)
    return out.reshape(_HALFV), jnp.transpose(x)


def _pooled_scores(s, x_t, batch, hist):
    V = s.shape[0]
    per_tile = batch // _NW
    n_blk = per_tile // _LANES

    mesh = plsc.VectorSubcoreMesh(
        core_axis_name="c", subcore_axis_name="s",
        num_cores=_NUM_CORES, num_subcores=_NUM_SUBCORES)

    @functools.partial(
        pl.kernel,
        out_type=jax.ShapeDtypeStruct((batch,), jnp.float32),
        mesh=mesh,
        compiler_params=pltpu.CompilerParams(needs_layout_passes=False),
        scratch_types=[
            pltpu.VMEM((V,), jnp.int32),
            pltpu.VMEM((hist, per_tile), jnp.int32),
            pltpu.VMEM((per_tile,), jnp.float32),
            pltpu.SemaphoreType.DMA,
        ],
    )
    def run(s_hbm, x_hbm, out_hbm, s_v, x_v, out_v, sem):
        wid = lax.axis_index("s") * _NUM_CORES + lax.axis_index("c")
        cp = pltpu.async_copy(s_hbm, s_v, sem)
        pltpu.sync_copy(x_hbm.at[:, pl.ds(wid * per_tile, per_tile)], x_v)
        cp.wait()

        unroll = 4
        assert hist % unroll == 0

        def step(j0, accs):
            new = list(accs)
            for u in range(unroll):
                j = j0 * unroll + u
                for rb in range(n_blk):
                    idx = x_v[j, pl.ds(rb * _LANES, _LANES)]
                    in_hi = lax.ge(idx, jnp.int32(_HALFV))
                    word = plsc.load_gather(
                        s_v, [lax.select(in_hi, idx - _HALFV, idx)])
                    bits = lax.select(
                        in_hi,
                        lax.bitwise_and(word, jnp.int32(-65536)),
                        lax.shift_left(word, 16))
                    new[rb] = new[rb] + plsc.bitcast(bits, jnp.float32)
            return tuple(new)

        accs = lax.fori_loop(
            0, hist // unroll, step,
            tuple(jnp.zeros((_LANES,), jnp.float32) for _ in range(n_blk)))
        for rb in range(n_blk):
            out_v[pl.ds(rb * _LANES, _LANES)] = accs[rb]
        pltpu.sync_copy(out_v, out_hbm.at[pl.ds(wid * per_tile, per_tile)])

    return run(s, x_t)


def kernel(x, emb, W, b):
    B, H = x.shape
    s, x_t = _scores(emb, W, b, x, H)
    pooled = _pooled_scores(s, x_t, B, H)
    return pooled.reshape(B, 1)


# speedup vs baseline: 1.0108x; 1.0108x over previous
import functools

import jax
import jax.numpy as jnp
from jax import lax
from jax.experimental import pallas as pl
from jax.experimental.pallas import tpu as pltpu
from jax.experimental.pallas import tpu_sc as plsc

_NUM_CORES = 2
_NUM_SUBCORES = 16
_NW = _NUM_CORES * _NUM_SUBCORES
_LANES = 16

_ROW_BLK = 6400
_NBH = 8
_HALFV = _NBH * _ROW_BLK


def _round_bf16_bits(prod):
    bits = lax.bitcast_convert_type(prod, jnp.int32)
    lsb = lax.bitwise_and(lax.shift_right_logical(bits, 16), 1)
    return lax.shift_right_logical(bits + 0x7FFF + lsb, 16)


def _scores(emb, W, b, x, hist):
    V, D = emb.shape
    B, H = x.shape
    assert V <= 2 * _HALFV and B % _NBH == 0
    xrows = B // _NBH
    inv = 1.0 / float(hist)

    def body(lo_ref, hi_ref, w_ref, b_ref, out_ref):
        dims = (((1,), (1,)), ((), ()))
        lo = lax.dot_general(w_ref[:], lo_ref[:], dims,
                             preferred_element_type=jnp.float32)
        hi = lax.dot_general(w_ref[:], hi_ref[:], dims,
                             preferred_element_type=jnp.float32)
        lo = lo * inv + b_ref[0] * inv
        hi = hi * inv + b_ref[0] * inv
        out_ref[0] = lax.bitwise_or(
            _round_bf16_bits(lo), lax.shift_left(_round_bf16_bits(hi), 16))

    out = pl.pallas_call(
        body,
        grid=(_NBH,),
        in_specs=[
            pl.BlockSpec((_ROW_BLK, D), lambda i: (i, 0)),
            pl.BlockSpec((_ROW_BLK, D), lambda i: (i + _NBH, 0)),
            pl.BlockSpec((1, D), lambda i: (0, 0)),
            pl.BlockSpec(memory_space=pltpu.SMEM),
        ],
        out_specs=pl.BlockSpec((1, 1, _ROW_BLK), lambda i: (i, 0, 0)),
        out_shape=jax.ShapeDtypeStruct((_NBH, 1, _ROW_BLK), jnp.int32),
    )(emb, emb, W, b)
    return out.reshape(_HALFV), jnp.transpose(x)


def _pooled_scores(s, x_t, batch, hist):
    V = s.shape[0]
    per_tile = batch // _NW
    n_blk = per_tile // _LANES

    mesh = plsc.VectorSubcoreMesh(
        core_axis_name="c", subcore_axis_name="s",
        num_cores=_NUM_CORES, num_subcores=_NUM_SUBCORES)

    @functools.partial(
        pl.kernel,
        out_type=jax.ShapeDtypeStruct((batch,), jnp.float32),
        mesh=mesh,
        compiler_params=pltpu.CompilerParams(needs_layout_passes=False),
        scratch_types=[
            pltpu.VMEM((V,), jnp.int32),
            pltpu.VMEM((hist, per_tile), jnp.int32),
            pltpu.VMEM((per_tile,), jnp.float32),
            pltpu.SemaphoreType.DMA,
        ],
    )
    def run(s_hbm, x_hbm, out_hbm, s_v, x_v, out_v, sem):
        wid = lax.axis_index("s") * _NUM_CORES + lax.axis_index("c")
        cp = pltpu.async_copy(s_hbm, s_v, sem)
        pltpu.sync_copy(x_hbm.at[:, pl.ds(wid * per_tile, per_tile)], x_v)
        cp.wait()

        unroll = 4
        assert hist % unroll == 0

        def step(j0, accs):
            new = list(accs)
            for u in range(unroll):
                j = j0 * unroll + u
                for rb in range(n_blk):
                    idx = x_v[j, pl.ds(rb * _LANES, _LANES)]
                    in_hi = lax.ge(idx, jnp.int32(_HALFV))
                    word = plsc.load_gather(
                        s_v, [lax.select(in_hi, idx - _HALFV, idx)])
                    bits = lax.select(
                        in_hi,
                        lax.bitwise_and(word, jnp.int32(-65536)),
                        lax.shift_left(word, 16))
                    new[rb] = new[rb] + plsc.bitcast(bits, jnp.float32)
            return tuple(new)

        accs = lax.fori_loop(
            0, hist // unroll, step,
            tuple(jnp.zeros((_LANES,), jnp.float32) for _ in range(n_blk)))
        for rb in range(n_blk):
            out_v[pl.ds(rb * _LANES, _LANES)] = accs[rb]
        pltpu.sync_copy(out_v, out_hbm.at[pl.ds(wid * per_tile, per_tile)])

    return run(s, x_t)


def kernel(x, emb, W, b):
    B, H = x.shape
    s, x_t = _scores(emb, W, b, x, H)
    pooled = _pooled_scores(s, x_t, B, H)
    return pooled.reshape(B, 1)


# speedup vs baseline: 1.0384x; 1.0273x over previous
import functools

import jax
import jax.numpy as jnp
from jax import lax
from jax.experimental import pallas as pl
from jax.experimental.pallas import tpu as pltpu
from jax.experimental.pallas import tpu_sc as plsc

_NUM_CORES = 2
_NUM_SUBCORES = 16
_NW = _NUM_CORES * _NUM_SUBCORES
_LANES = 16

_ROW_BLK = 6400
_NBH = 8
_HALFV = _NBH * _ROW_BLK


def _round_bf16_bits(prod):
    bits = lax.bitcast_convert_type(prod, jnp.int32)
    lsb = lax.bitwise_and(lax.shift_right_logical(bits, 16), 1)
    return lax.shift_right_logical(bits + 0x7FFF + lsb, 16)


def _scores(emb, W, b, x, hist):
    V, D = emb.shape
    B, H = x.shape
    assert V <= 2 * _HALFV and B % _NBH == 0
    xrows = B // _NBH
    inv = 1.0 / float(hist)

    def body(lo_ref, hi_ref, w_ref, b_ref, out_ref):
        dims = (((1,), (1,)), ((), ()))
        lo = lax.dot_general(w_ref[:], lo_ref[:], dims,
                             preferred_element_type=jnp.float32)
        hi = lax.dot_general(w_ref[:], hi_ref[:], dims,
                             preferred_element_type=jnp.float32)
        lo = lo * inv + b_ref[0] * inv
        hi = hi * inv + b_ref[0] * inv
        out_ref[0] = lax.bitwise_or(
            _round_bf16_bits(lo), lax.shift_left(_round_bf16_bits(hi), 16))

    out = pl.pallas_call(
        body,
        grid=(_NBH,),
        in_specs=[
            pl.BlockSpec((_ROW_BLK, D), lambda i: (i, 0)),
            pl.BlockSpec((_ROW_BLK, D), lambda i: (i + _NBH, 0)),
            pl.BlockSpec((1, D), lambda i: (0, 0)),
            pl.BlockSpec(memory_space=pltpu.SMEM),
        ],
        out_specs=pl.BlockSpec((1, 1, _ROW_BLK), lambda i: (i, 0, 0)),
        out_shape=jax.ShapeDtypeStruct((_NBH, 1, _ROW_BLK), jnp.int32),
    )(emb, emb, W, b)
    return out.reshape(_HALFV), jnp.transpose(x)


def _pooled_scores(s, x_t, batch, hist):
    V = s.shape[0]
    per_tile = batch // _NW
    n_blk = per_tile // _LANES

    mesh = plsc.VectorSubcoreMesh(
        core_axis_name="c", subcore_axis_name="s",
        num_cores=_NUM_CORES, num_subcores=_NUM_SUBCORES)

    @functools.partial(
        pl.kernel,
        out_type=jax.ShapeDtypeStruct((batch,), jnp.float32),
        mesh=mesh,
        compiler_params=pltpu.CompilerParams(needs_layout_passes=False),
        scratch_types=[
            pltpu.VMEM((V,), jnp.int32),
            pltpu.VMEM((hist, per_tile), jnp.int32),
            pltpu.VMEM((per_tile,), jnp.float32),
            pltpu.SemaphoreType.DMA,
        ],
    )
    def run(s_hbm, x_hbm, out_hbm, s_v, x_v, out_v, sem):
        wid = lax.axis_index("s") * _NUM_CORES + lax.axis_index("c")
        cp = pltpu.async_copy(s_hbm, s_v, sem)
        pltpu.sync_copy(x_hbm.at[:, pl.ds(wid * per_tile, per_tile)], x_v)
        cp.wait()

        unroll = 2
        assert hist % unroll == 0

        def step(j0, accs):
            new = list(accs)
            for u in range(unroll):
                j = j0 * unroll + u
                for rb in range(n_blk):
                    idx = x_v[j, pl.ds(rb * _LANES, _LANES)]
                    in_hi = lax.ge(idx, jnp.int32(_HALFV))
                    word = plsc.load_gather(
                        s_v, [lax.select(in_hi, idx - _HALFV, idx)])
                    bits = lax.select(
                        in_hi,
                        lax.bitwise_and(word, jnp.int32(-65536)),
                        lax.shift_left(word, 16))
                    new[rb] = new[rb] + plsc.bitcast(bits, jnp.float32)
            return tuple(new)

        accs = lax.fori_loop(
            0, hist // unroll, step,
            tuple(jnp.zeros((_LANES,), jnp.float32) for _ in range(n_blk)))
        for rb in range(n_blk):
            out_v[pl.ds(rb * _LANES, _LANES)] = accs[rb]
        pltpu.sync_copy(out_v, out_hbm.at[pl.ds(wid * per_tile, per_tile)])

    return run(s, x_t)


def kernel(x, emb, W, b):
    B, H = x.shape
    s, x_t = _scores(emb, W, b, x, H)
    pooled = _pooled_scores(s, x_t, B, H)
    return pooled.reshape(B, 1)
